# baseline (speedup 1.0000x reference)
import functools

import jax
import jax.numpy as jnp
import numpy as np
from jax import lax
from jax.experimental import pallas as pl
from jax.experimental.pallas import tpu as pltpu
from jax.experimental.pallas import tpu_sc as plsc

NC, NS, L = 2, 16, 16
NW = NC * NS

NELEM = 2 * 4096 * 2048
_PER_LOW = 0.1 * 0.01
_PER_HIGH = 99.9 * 0.01
_lower_k = int(_PER_LOW * NELEM)
K_LO = _lower_k if _lower_k > 0 else 1
K_HI = int(_PER_HIGH * NELEM)

ROWS = 8192
COLS = 2048
ROWS_PT = ROWS // NW
CHUNK_R = 8
CHUNK = CHUNK_R * COLS
UNROLL = 8

H1 = 65536
H2 = H1 + L


@functools.cache
def _build_sc_kernels():
    mesh = plsc.VectorSubcoreMesh(
        core_axis_name="c", subcore_axis_name="s",
        num_cores=NC, num_subcores=NS,
    )
    cparams = pltpu.CompilerParams(
        needs_layout_passes=False, use_tc_tiling_on_sc=True)

    def _prime(x_hbm, base, b0, b1, s0, s1):
        pltpu.async_copy(x_hbm.at[pl.ds(base, CHUNK_R), :], b0, s0)
        pltpu.async_copy(x_hbm.at[pl.ds(base + CHUNK_R, CHUNK_R), :], b1, s1)

    def _scan_chunks(x_hbm, base, n_pairs, b0, b1, s0, s1, process):
        def pair(p, _):
            r0 = base + 2 * p * CHUNK_R
            pltpu.make_async_copy(
                x_hbm.at[pl.ds(base, CHUNK_R), :], b0, s0).wait()
            process(b0)

            @pl.when(p < n_pairs - 1)
            def _():
                pltpu.async_copy(
                    x_hbm.at[pl.ds(r0 + 2 * CHUNK_R, CHUNK_R), :], b0, s0)

            pltpu.make_async_copy(
                x_hbm.at[pl.ds(base, CHUNK_R), :], b1, s1).wait()
            process(b1)

            @pl.when(p < n_pairs - 1)
            def _():
                pltpu.async_copy(
                    x_hbm.at[pl.ds(r0 + 3 * CHUNK_R, CHUNK_R), :], b1, s1)

            return 0

        lax.fori_loop(0, n_pairs, pair, 0)

    @functools.partial(
        pl.kernel,
        out_type=jax.ShapeDtypeStruct((NW, H1), jnp.int32),
        mesh=mesh,
        compiler_params=cparams,
        scratch_types=[
            pltpu.VMEM((CHUNK_R, COLS), jnp.float32),
            pltpu.VMEM((CHUNK_R, COLS), jnp.float32),
            pltpu.VMEM((H1,), jnp.int32),
            pltpu.SemaphoreType.DMA,
            pltpu.SemaphoreType.DMA,
        ],
    )
    def _sc_pass1(x_hbm, out_hbm, b0, b1, hist, s0, s1):
        wid = lax.axis_index("s") * NC + lax.axis_index("c")
        base = wid * ROWS_PT
        _prime(x_hbm, base, b0, b1, s0, s1)

        zeros = jnp.zeros((L,), jnp.int32)
        def zbody(i, _):
            for u_ in range(UNROLL):
                hist[pl.ds(i * (L * UNROLL) + u_ * L, L)] = zeros
            return 0
        lax.fori_loop(0, H1 // (L * UNROLL), zbody, 0)

        ones = jnp.ones((L,), jnp.int32)

        def process(buf):
            @functools.partial(
                plsc.parallel_loop, 0, CHUNK // L, unroll=UNROLL)
            def vec_body(i):
                r = lax.shift_right_logical(i, 7)
                c = lax.bitwise_and(i, 127) * L
                v = buf[r, pl.ds(c, L)]
                u = plsc.bitcast(v, jnp.int32)
                b = lax.shift_right_logical(u, 16)
                plsc.addupdate_scatter(hist, [b], ones)

        _scan_chunks(x_hbm, base, ROWS_PT // CHUNK_R // 2, b0, b1, s0, s1,
                     process)
        pltpu.sync_copy(hist, out_hbm.at[wid])

    @functools.partial(
        pl.kernel,
        out_type=jax.ShapeDtypeStruct((NW, H2), jnp.int32),
        mesh=mesh,
        compiler_params=cparams,
        scratch_types=[
            pltpu.VMEM((CHUNK_R, COLS), jnp.float32),
            pltpu.VMEM((CHUNK_R, COLS), jnp.float32),
            pltpu.VMEM((H2,), jnp.int32),
            pltpu.VMEM((8, L), jnp.int32),
            pltpu.SemaphoreType.DMA,
            pltpu.SemaphoreType.DMA,
        ],
    )
    def _sc_pass2(x_hbm, pf_hbm, out_hbm, b0, b1, hist, pfv, s0, s1):
        wid = lax.axis_index("s") * NC + lax.axis_index("c")
        grp = lax.bitwise_and(wid, 1)
        base = lax.shift_right_logical(wid, 1) * (2 * ROWS_PT)
        _prime(x_hbm, base, b0, b1, s0, s1)
        pltpu.sync_copy(pf_hbm, pfv)
        gv = jnp.zeros((L,), jnp.int32) + grp
        mypfx = jnp.where(gv == 0, pfv[0], pfv[1])

        zeros = jnp.zeros((L,), jnp.int32)
        def zbody(i, _):
            for u_ in range(UNROLL):
                hist[pl.ds(i * (L * UNROLL) + u_ * L, L)] = zeros
            return 0
        lax.fori_loop(0, H1 // (L * UNROLL), zbody, 0)
        hist[pl.ds(H1, L)] = zeros

        ones = jnp.ones((L,), jnp.int32)
        cmask = jnp.full((L,), np.int32(0xFFFF))
        cdump = jnp.full((L,), np.int32(H1))

        def process(buf):
            @functools.partial(
                plsc.parallel_loop, 0, CHUNK // L, unroll=UNROLL)
            def vec_body(i):
                r = lax.shift_right_logical(i, 7)
                c = lax.bitwise_and(i, 127) * L
                v = buf[r, pl.ds(c, L)]
                u = plsc.bitcast(v, jnp.int32)
                hi = lax.shift_right_logical(u, 16)
                low = lax.bitwise_and(u, cmask)
                b = jnp.where(hi == mypfx, low, cdump)
                plsc.addupdate_scatter(hist, [b], ones)

        _scan_chunks(x_hbm, base, ROWS_PT // CHUNK_R, b0, b1, s0, s1,
                     process)
        pltpu.sync_copy(hist, out_hbm.at[wid])

    return _sc_pass1, _sc_pass2


def _scan2d(t, suffix=False):
    r, c = t.shape
    s = t
    sh = 1
    while sh < c:
        if suffix:
            shifted = jnp.concatenate(
                [s[:, sh:], jnp.zeros((r, sh), jnp.int32)], axis=1)
        else:
            shifted = jnp.concatenate(
                [jnp.zeros((r, sh), jnp.int32), s[:, : c - sh]], axis=1)
        s = s + shifted
        sh *= 2
    rt = s[:, 0:1] if suffix else s[:, c - 1 : c]
    o = rt
    sh = 1
    while sh < r:
        if suffix:
            shifted = jnp.concatenate(
                [o[sh:, :], jnp.zeros((sh, 1), jnp.int32)], axis=0)
        else:
            shifted = jnp.concatenate(
                [jnp.zeros((sh, 1), jnp.int32), o[: r - sh, :]], axis=0)
        o = o + shifted
        sh *= 2
    return s + (o - rt)


def _glue1_body(h_ref, o_ref):
    h = h_ref[...]
    t = jnp.sum(jnp.reshape(h, (NW, 512, 128)), axis=0)
    fi = (lax.broadcasted_iota(jnp.int32, (512, 128), 0) * 128
          + lax.broadcasted_iota(jnp.int32, (512, 128), 1))
    neg = fi >= 32768
    tpos = jnp.where(neg, 0, t)
    tneg = jnp.where(neg, t, 0)
    total_neg = jnp.sum(tneg)
    C = jnp.where(neg, _scan2d(tneg, suffix=True), _scan2d(tpos) + total_neg)

    def pick(k):
        b_ord = jnp.sum((C < k).astype(jnp.int32))
        cb = jnp.max(jnp.where(C < k, C, 0))
        raw = jnp.where(b_ord < 32768, 65535 - b_ord, b_ord - 32768)
        return raw, cb

    p_lo, cb_lo = pick(K_LO)
    p_hi, cb_hi = pick(K_HI)
    z = jnp.zeros((L,), jnp.int32)
    o_ref[...] = jnp.stack([
        jnp.full((L,), p_lo), jnp.full((L,), p_hi),
        jnp.full((L,), cb_lo), jnp.full((L,), cb_hi),
        z, z, z, z,
    ])


_glue1 = pl.pallas_call(
    _glue1_body, out_shape=jax.ShapeDtypeStruct((8, L), jnp.int32)
)


def _glue2_body(h_ref, pf_ref, w_ref, o_ref):
    h = h_ref[...]
    hh = jnp.reshape(h[:, :H1], (NW // 2, 2, 512, 128))
    t_lo = jnp.sum(hh[:, 0], axis=0)
    t_hi = jnp.sum(hh[:, 1], axis=0)

    pfx_lo = pf_ref[0, 0]
    pfx_hi = pf_ref[1, 0]
    cb_lo = pf_ref[2, 0]
    cb_hi = pf_ref[3, 0]
    t_hi = jnp.where(pfx_lo == pfx_hi, t_lo, t_hi)

    def pick(t, pfx, kp):
        is_neg = lax.shift_right_logical(pfx, 15) == 1
        C = jnp.where(is_neg, _scan2d(t, suffix=True), _scan2d(t))
        b_ord = jnp.sum((C < kp).astype(jnp.int32))
        raw = jnp.where(is_neg, 65535 - b_ord, b_ord)
        return lax.shift_left(pfx, 16) | raw

    key_lo = pick(t_lo, pfx_lo, K_LO - cb_lo)
    key_hi = pick(t_hi, pfx_hi, K_HI - cb_hi)
    lower_val = lax.bitcast_convert_type(key_lo, jnp.float32)
    upper_val = lax.bitcast_convert_type(key_hi, jnp.float32)

    w = w_ref[...]
    n = jnp.float32(w.size)
    w_abs_mean = jnp.sum(jnp.abs(w)) / n
    w_std = jnp.sqrt(jnp.sum(w * w) / n)
    w_clip = jnp.float32(-12.8) * w_abs_mean + jnp.float32(12.68) * w_std

    row = lax.broadcasted_iota(jnp.int32, (8, 128), 0)
    col = lax.broadcasted_iota(jnp.int32, (8, 128), 1)
    vals = jnp.where(
        col == 0, upper_val, jnp.where(col == 1, lower_val, w_clip)
    )
    o_ref[...] = jnp.where((row == 0) & (col < 3), vals, 0.0)


_glue2 = pl.pallas_call(
    _glue2_body, out_shape=jax.ShapeDtypeStruct((8, 128), jnp.float32)
)


def kernel(x, weight):
    _sc_pass1, _sc_pass2 = _build_sc_kernels()
    xf = jnp.reshape(x, (ROWS, COLS))
    h1 = _sc_pass1(xf)
    pf1 = _glue1(h1)
    h2 = _sc_pass2(xf, pf1)
    o = _glue2(h2, pf1, weight)
    return o[0, :3]


# speedup vs baseline: 1.0868x; 1.0868x over previous
import functools

import jax
import jax.numpy as jnp
import numpy as np
from jax import lax
from jax.experimental import pallas as pl
from jax.experimental.pallas import tpu as pltpu
from jax.experimental.pallas import tpu_sc as plsc

NC, NS, L = 2, 16, 16
NW = NC * NS

NELEM = 2 * 4096 * 2048
_PER_LOW = 0.1 * 0.01
_PER_HIGH = 99.9 * 0.01
_lower_k = int(_PER_LOW * NELEM)
K_LO = _lower_k if _lower_k > 0 else 1
K_HI = int(_PER_HIGH * NELEM)

ROWS = 8192
COLS = 2048
ROWS_PT = ROWS // NW
CHUNK_R = 8
CHUNK = CHUNK_R * COLS
N_CHUNKS = ROWS_PT // CHUNK_R
N_PAIRS = N_CHUNKS // 2
UNROLL = 8
ITERS = CHUNK // (L * UNROLL)

B1, B2, B3 = 13, 10, 9
H1 = 1 << B1


@functools.cache
def _build_sc_kernels():
    mesh = plsc.VectorSubcoreMesh(
        core_axis_name="c", subcore_axis_name="s",
        num_cores=NC, num_subcores=NS,
    )

    def _prime(x_hbm, base, b0, b1, s0, s1):
        pltpu.async_copy(x_hbm.at[pl.ds(base, CHUNK_R), :], b0, s0)
        pltpu.async_copy(x_hbm.at[pl.ds(base + CHUNK_R, CHUNK_R), :], b1, s1)

    def _scan_chunks(x_hbm, base, b0, b1, s0, s1, process):
        def pair(p, _):
            r0 = base + 2 * p * CHUNK_R
            pltpu.make_async_copy(
                x_hbm.at[pl.ds(base, CHUNK_R), :], b0, s0).wait()
            process(b0)

            @pl.when(p < N_PAIRS - 1)
            def _():
                pltpu.async_copy(
                    x_hbm.at[pl.ds(r0 + 2 * CHUNK_R, CHUNK_R), :], b0, s0)

            pltpu.make_async_copy(
                x_hbm.at[pl.ds(base, CHUNK_R), :], b1, s1).wait()
            process(b1)

            @pl.when(p < N_PAIRS - 1)
            def _():
                pltpu.async_copy(
                    x_hbm.at[pl.ds(r0 + 3 * CHUNK_R, CHUNK_R), :], b1, s1)

            return 0

        lax.fori_loop(0, N_PAIRS, pair, 0)

    @functools.partial(
        pl.kernel,
        out_type=jax.ShapeDtypeStruct((NW, H1), jnp.int32),
        mesh=mesh,
        compiler_params=pltpu.CompilerParams(
            needs_layout_passes=False, use_tc_tiling_on_sc=True),
        scratch_types=[
            pltpu.VMEM((CHUNK_R, COLS), jnp.float32),
            pltpu.VMEM((CHUNK_R, COLS), jnp.float32),
            pltpu.VMEM((H1,), jnp.int32),
            pltpu.SemaphoreType.DMA,
            pltpu.SemaphoreType.DMA,
        ],
    )
    def _sc_pass1(x_hbm, out_hbm, b0, b1, hist, s0, s1):
        wid = lax.axis_index("s") * NC + lax.axis_index("c")
        base = wid * ROWS_PT
        _prime(x_hbm, base, b0, b1, s0, s1)

        zeros = jnp.zeros((L,), jnp.int32)
        def zbody(i, _):
            for u_ in range(UNROLL):
                hist[pl.ds(i * (L * UNROLL) + u_ * L, L)] = zeros
            return 0
        lax.fori_loop(0, H1 // (L * UNROLL), zbody, 0)

        ones = jnp.ones((L,), jnp.int32)

        def process(buf):
            @functools.partial(
                plsc.parallel_loop, 0, CHUNK // L, unroll=UNROLL)
            def vec_body(i):
                r = lax.shift_right_logical(i, 7)
                c = lax.bitwise_and(i, 127) * L
                v = buf[r, pl.ds(c, L)]
                u = plsc.bitcast(v, jnp.int32)
                b = lax.shift_right_logical(u, 32 - B1)
                plsc.addupdate_scatter(hist, [b], ones)

        _scan_chunks(x_hbm, base, b0, b1, s0, s1, process)
        pltpu.sync_copy(hist, out_hbm.at[wid])

    def _make_refine(hi_shift, lo_shift, bits, hsz):
        n = 1 << bits

        @functools.partial(
            pl.kernel,
            out_type=jax.ShapeDtypeStruct((NW, hsz), jnp.int32),
            mesh=mesh,
            compiler_params=pltpu.CompilerParams(
                needs_layout_passes=False, use_tc_tiling_on_sc=True),
            scratch_types=[
                pltpu.VMEM((CHUNK_R, COLS), jnp.float32),
                pltpu.VMEM((CHUNK_R, COLS), jnp.float32),
                pltpu.VMEM((hsz,), jnp.int32),
                pltpu.VMEM((8, L), jnp.int32),
                pltpu.SemaphoreType.DMA,
                pltpu.SemaphoreType.DMA,
            ],
        )
        def _sc_refine(x_hbm, pf_hbm, out_hbm, b0, b1, hist, pfv, s0, s1):
            wid = lax.axis_index("s") * NC + lax.axis_index("c")
            base = wid * ROWS_PT

            _prime(x_hbm, base, b0, b1, s0, s1)
            pltpu.sync_copy(pf_hbm, pfv)
            pfx_lo = pfv[0]
            pfx_hi = pfv[1]

            zeros = jnp.zeros((L,), jnp.int32)
            def zbody(i, _):
                hist[pl.ds(i * L, L)] = zeros
                return 0
            lax.fori_loop(0, hsz // L, zbody, 0)

            ones = jnp.ones((L,), jnp.int32)
            cmask = jnp.full((L,), np.int32(n - 1))
            coffs = jnp.full((L,), np.int32(n))
            cdump = jnp.full((L,), np.int32(2 * n))

            def process(buf):
                @functools.partial(
                    plsc.parallel_loop, 0, CHUNK // L, unroll=UNROLL)
                def vec_body(i):
                    r = lax.shift_right_logical(i, 7)
                    c = lax.bitwise_and(i, 127) * L
                    v = buf[r, pl.ds(c, L)]
                    u = plsc.bitcast(v, jnp.int32)
                    hi = lax.shift_right_logical(u, hi_shift)
                    low = lax.bitwise_and(
                        lax.shift_right_logical(u, lo_shift), cmask
                    )
                    b = jnp.where(
                        hi == pfx_lo,
                        low,
                        jnp.where(hi == pfx_hi, low + coffs, cdump),
                    )
                    plsc.addupdate_scatter(hist, [b], ones)

            _scan_chunks(x_hbm, base, b0, b1, s0, s1, process)
            pltpu.sync_copy(hist, out_hbm.at[wid])

        return _sc_refine

    return (_sc_pass1,
            _make_refine(32 - B1, 32 - B1 - B2, B2, 2 * (1 << B2) + L),
            _make_refine(B3, 0, B3, 2 * (1 << B3) + L))


def _scan2d(t, suffix=False):
    r, c = t.shape
    s = t
    sh = 1
    while sh < c:
        if suffix:
            shifted = jnp.concatenate(
                [s[:, sh:], jnp.zeros((r, sh), jnp.int32)], axis=1)
        else:
            shifted = jnp.concatenate(
                [jnp.zeros((r, sh), jnp.int32), s[:, : c - sh]], axis=1)
        s = s + shifted
        sh *= 2
    rt = s[:, 0:1] if suffix else s[:, c - 1 : c]
    o = rt
    sh = 1
    while sh < r:
        if suffix:
            shifted = jnp.concatenate(
                [o[sh:, :], jnp.zeros((sh, 1), jnp.int32)], axis=0)
        else:
            shifted = jnp.concatenate(
                [jnp.zeros((sh, 1), jnp.int32), o[: r - sh, :]], axis=0)
        o = o + shifted
        sh *= 2
    return s + (o - rt)


def _glue1_body(h_ref, o_ref):
    R1 = H1 // 128
    h = h_ref[...]
    t = jnp.sum(jnp.reshape(h, (NW, R1, 128)), axis=0)
    fi = (lax.broadcasted_iota(jnp.int32, (R1, 128), 0) * 128
          + lax.broadcasted_iota(jnp.int32, (R1, 128), 1))
    half = H1 // 2
    neg = fi >= half
    tpos = jnp.where(neg, 0, t)
    tneg = jnp.where(neg, t, 0)
    total_neg = jnp.sum(tneg)
    C = jnp.where(neg, _scan2d(tneg, suffix=True), _scan2d(tpos) + total_neg)

    def pick(k):
        b_ord = jnp.sum((C < k).astype(jnp.int32))
        cb = jnp.max(jnp.where(C < k, C, 0))
        raw = jnp.where(b_ord < half, H1 - 1 - b_ord, b_ord - half)
        return raw, cb

    p_lo, cb_lo = pick(K_LO)
    p_hi, cb_hi = pick(K_HI)
    z = jnp.zeros((L,), jnp.int32)
    o_ref[...] = jnp.stack([
        jnp.full((L,), p_lo), jnp.full((L,), p_hi),
        jnp.full((L,), cb_lo), jnp.full((L,), cb_hi),
        z, z, z, z,
    ])


_glue1 = pl.pallas_call(
    _glue1_body, out_shape=jax.ShapeDtypeStruct((8, L), jnp.int32)
)


def _region_pick(cnt, is_neg, kp, n):
    C = jnp.where(is_neg, _scan2d(cnt, suffix=True), _scan2d(cnt))
    b_ord = jnp.sum((C < kp).astype(jnp.int32))
    cb = jnp.max(jnp.where(C < kp, C, 0))
    raw = jnp.where(is_neg, n - 1 - b_ord, b_ord)
    return raw, cb


def _refine_pick(h, pf, sign_shift, bits):
    n = 1 << bits
    R = n // 128
    t = jnp.sum(
        jnp.reshape(h[:, : 2 * n], (NW, 2, R, 128)), axis=0)
    r_lo = t[0]
    r_hi = t[1]

    pfx_lo = pf[0, 0]
    pfx_hi = pf[1, 0]
    cb1_lo = pf[2, 0]
    cb1_hi = pf[3, 0]
    r_hi = jnp.where(pfx_lo == pfx_hi, r_lo, r_hi)
    neg_lo = lax.shift_right_logical(pfx_lo, sign_shift) == 1
    neg_hi = lax.shift_right_logical(pfx_hi, sign_shift) == 1

    b_lo, cb2_lo = _region_pick(r_lo, neg_lo, K_LO - cb1_lo, n)
    b_hi, cb2_hi = _region_pick(r_hi, neg_hi, K_HI - cb1_hi, n)
    npfx_lo = lax.shift_left(pfx_lo, bits) | b_lo
    npfx_hi = lax.shift_left(pfx_hi, bits) | b_hi
    return npfx_lo, npfx_hi, cb1_lo + cb2_lo, cb1_hi + cb2_hi


def _glue2_body(h_ref, pf_ref, o_ref):
    npfx_lo, npfx_hi, ncb_lo, ncb_hi = _refine_pick(
        h_ref[...], pf_ref[...], sign_shift=B1 - 1, bits=B2)
    z = jnp.zeros((L,), jnp.int32)
    o_ref[...] = jnp.stack([
        jnp.full((L,), npfx_lo), jnp.full((L,), npfx_hi),
        jnp.full((L,), ncb_lo), jnp.full((L,), ncb_hi),
        z, z, z, z,
    ])


_glue2 = pl.pallas_call(
    _glue2_body, out_shape=jax.ShapeDtypeStruct((8, L), jnp.int32)
)


def _glue3_body(h_ref, pf_ref, w_ref, o_ref):
    key_lo, key_hi, _, _ = _refine_pick(
        h_ref[...], pf_ref[...], sign_shift=B1 + B2 - 1, bits=B3)

    lower_val = lax.bitcast_convert_type(key_lo, jnp.float32)
    upper_val = lax.bitcast_convert_type(key_hi, jnp.float32)

    w = w_ref[...]
    n = jnp.float32(w.size)
    w_abs_mean = jnp.sum(jnp.abs(w)) / n
    w_std = jnp.sqrt(jnp.sum(w * w) / n)
    w_clip = jnp.float32(-12.8) * w_abs_mean + jnp.float32(12.68) * w_std

    row = lax.broadcasted_iota(jnp.int32, (8, 128), 0)
    col = lax.broadcasted_iota(jnp.int32, (8, 128), 1)
    vals = jnp.where(
        col == 0, upper_val, jnp.where(col == 1, lower_val, w_clip)
    )
    o_ref[...] = jnp.where((row == 0) & (col < 3), vals, 0.0)


_glue3 = pl.pallas_call(
    _glue3_body, out_shape=jax.ShapeDtypeStruct((8, 128), jnp.float32)
)


def kernel(x, weight):
    _sc_pass1, _sc_pass2, _sc_pass3 = _build_sc_kernels()
    xf = jnp.reshape(x, (ROWS, COLS))
    h1 = _sc_pass1(xf)
    pf1 = _glue1(h1)
    h2 = _sc_pass2(xf, pf1)
    pf2 = _glue2(h2, pf1)
    h3 = _sc_pass3(xf, pf2)
    o = _glue3(h3, pf2, weight)
    return o[0, :3]


# speedup vs baseline: 1.1598x; 1.0672x over previous
import functools

import jax
import jax.numpy as jnp
import numpy as np
from jax import lax
from jax.experimental import pallas as pl
from jax.experimental.pallas import tpu as pltpu
from jax.experimental.pallas import tpu_sc as plsc

NC, NS, L = 2, 16, 16
NW = NC * NS

NELEM = 2 * 4096 * 2048
_PER_LOW = 0.1 * 0.01
_PER_HIGH = 99.9 * 0.01
_lower_k = int(_PER_LOW * NELEM)
K_LO = _lower_k if _lower_k > 0 else 1
K_HI = int(_PER_HIGH * NELEM)

ROWS = 8192
COLS = 2048
ROWS_PT = ROWS // NW
CHUNK_R = 16
CHUNK = CHUNK_R * COLS
N_CHUNKS = ROWS_PT // CHUNK_R
N_PAIRS = N_CHUNKS // 2
UNROLL = 8
ITERS = CHUNK // (L * UNROLL)

B1, B2, B3 = 13, 10, 9
H1 = 1 << B1


@functools.cache
def _build_sc_kernels():
    mesh = plsc.VectorSubcoreMesh(
        core_axis_name="c", subcore_axis_name="s",
        num_cores=NC, num_subcores=NS,
    )

    def _prime(x_hbm, base, b0, b1, s0, s1):
        pltpu.async_copy(x_hbm.at[pl.ds(base, CHUNK_R), :], b0, s0)
        pltpu.async_copy(x_hbm.at[pl.ds(base + CHUNK_R, CHUNK_R), :], b1, s1)

    def _scan_chunks(x_hbm, base, b0, b1, s0, s1, process):
        def pair(p, _):
            r0 = base + 2 * p * CHUNK_R
            pltpu.make_async_copy(
                x_hbm.at[pl.ds(base, CHUNK_R), :], b0, s0).wait()
            process(b0)

            @pl.when(p < N_PAIRS - 1)
            def _():
                pltpu.async_copy(
                    x_hbm.at[pl.ds(r0 + 2 * CHUNK_R, CHUNK_R), :], b0, s0)

            pltpu.make_async_copy(
                x_hbm.at[pl.ds(base, CHUNK_R), :], b1, s1).wait()
            process(b1)

            @pl.when(p < N_PAIRS - 1)
            def _():
                pltpu.async_copy(
                    x_hbm.at[pl.ds(r0 + 3 * CHUNK_R, CHUNK_R), :], b1, s1)

            return 0

        lax.fori_loop(0, N_PAIRS, pair, 0)

    @functools.partial(
        pl.kernel,
        out_type=jax.ShapeDtypeStruct((NW, H1), jnp.int32),
        mesh=mesh,
        compiler_params=pltpu.CompilerParams(
            needs_layout_passes=False, use_tc_tiling_on_sc=True),
        scratch_types=[
            pltpu.VMEM((CHUNK_R, COLS), jnp.float32),
            pltpu.VMEM((CHUNK_R, COLS), jnp.float32),
            pltpu.VMEM((H1,), jnp.int32),
            pltpu.SemaphoreType.DMA,
            pltpu.SemaphoreType.DMA,
        ],
    )
    def _sc_pass1(x_hbm, out_hbm, b0, b1, hist, s0, s1):
        wid = lax.axis_index("s") * NC + lax.axis_index("c")
        base = wid * ROWS_PT
        _prime(x_hbm, base, b0, b1, s0, s1)

        zeros = jnp.zeros((L,), jnp.int32)
        def zbody(i, _):
            for u_ in range(UNROLL):
                hist[pl.ds(i * (L * UNROLL) + u_ * L, L)] = zeros
            return 0
        lax.fori_loop(0, H1 // (L * UNROLL), zbody, 0)

        ones = jnp.ones((L,), jnp.int32)

        def process(buf):
            @functools.partial(
                plsc.parallel_loop, 0, CHUNK // L, unroll=UNROLL)
            def vec_body(i):
                r = lax.shift_right_logical(i, 7)
                c = lax.bitwise_and(i, 127) * L
                v = buf[r, pl.ds(c, L)]
                u = plsc.bitcast(v, jnp.int32)
                b = lax.shift_right_logical(u, 32 - B1)
                plsc.addupdate_scatter(hist, [b], ones)

        _scan_chunks(x_hbm, base, b0, b1, s0, s1, process)
        pltpu.sync_copy(hist, out_hbm.at[wid])

    def _make_refine(hi_shift, lo_shift, bits, hsz):
        n = 1 << bits

        @functools.partial(
            pl.kernel,
            out_type=jax.ShapeDtypeStruct((NW, hsz), jnp.int32),
            mesh=mesh,
            compiler_params=pltpu.CompilerParams(
                needs_layout_passes=False, use_tc_tiling_on_sc=True),
            scratch_types=[
                pltpu.VMEM((CHUNK_R, COLS), jnp.float32),
                pltpu.VMEM((CHUNK_R, COLS), jnp.float32),
                pltpu.VMEM((hsz,), jnp.int32),
                pltpu.VMEM((8, L), jnp.int32),
                pltpu.SemaphoreType.DMA,
                pltpu.SemaphoreType.DMA,
            ],
        )
        def _sc_refine(x_hbm, pf_hbm, out_hbm, b0, b1, hist, pfv, s0, s1):
            wid = lax.axis_index("s") * NC + lax.axis_index("c")
            base = wid * ROWS_PT

            _prime(x_hbm, base, b0, b1, s0, s1)
            pltpu.sync_copy(pf_hbm, pfv)
            pfx_lo = pfv[0]
            pfx_hi = pfv[1]

            zeros = jnp.zeros((L,), jnp.int32)
            def zbody(i, _):
                hist[pl.ds(i * L, L)] = zeros
                return 0
            lax.fori_loop(0, hsz // L, zbody, 0)

            ones = jnp.ones((L,), jnp.int32)
            cmask = jnp.full((L,), np.int32(n - 1))
            coffs = jnp.full((L,), np.int32(n))
            cdump = jnp.full((L,), np.int32(2 * n))

            def process(buf):
                @functools.partial(
                    plsc.parallel_loop, 0, CHUNK // L, unroll=UNROLL)
                def vec_body(i):
                    r = lax.shift_right_logical(i, 7)
                    c = lax.bitwise_and(i, 127) * L
                    v = buf[r, pl.ds(c, L)]
                    u = plsc.bitcast(v, jnp.int32)
                    hi = lax.shift_right_logical(u, hi_shift)
                    low = lax.bitwise_and(
                        lax.shift_right_logical(u, lo_shift), cmask
                    )
                    b = jnp.where(
                        hi == pfx_lo,
                        low,
                        jnp.where(hi == pfx_hi, low + coffs, cdump),
                    )
                    plsc.addupdate_scatter(hist, [b], ones)

            _scan_chunks(x_hbm, base, b0, b1, s0, s1, process)
            pltpu.sync_copy(hist, out_hbm.at[wid])

        return _sc_refine

    return (_sc_pass1,
            _make_refine(32 - B1, 32 - B1 - B2, B2, 2 * (1 << B2) + L),
            _make_refine(B3, 0, B3, 2 * (1 << B3) + L))


def _scan2d(t, suffix=False):
    r, c = t.shape
    s = t
    sh = 1
    while sh < c:
        if suffix:
            shifted = jnp.concatenate(
                [s[:, sh:], jnp.zeros((r, sh), jnp.int32)], axis=1)
        else:
            shifted = jnp.concatenate(
                [jnp.zeros((r, sh), jnp.int32), s[:, : c - sh]], axis=1)
        s = s + shifted
        sh *= 2
    rt = s[:, 0:1] if suffix else s[:, c - 1 : c]
    o = rt
    sh = 1
    while sh < r:
        if suffix:
            shifted = jnp.concatenate(
                [o[sh:, :], jnp.zeros((sh, 1), jnp.int32)], axis=0)
        else:
            shifted = jnp.concatenate(
                [jnp.zeros((sh, 1), jnp.int32), o[: r - sh, :]], axis=0)
        o = o + shifted
        sh *= 2
    return s + (o - rt)


def _glue1_body(h_ref, o_ref):
    R1 = H1 // 128
    h = h_ref[...]
    t = jnp.sum(jnp.reshape(h, (NW, R1, 128)), axis=0)
    fi = (lax.broadcasted_iota(jnp.int32, (R1, 128), 0) * 128
          + lax.broadcasted_iota(jnp.int32, (R1, 128), 1))
    half = H1 // 2
    neg = fi >= half
    tpos = jnp.where(neg, 0, t)
    tneg = jnp.where(neg, t, 0)
    total_neg = jnp.sum(tneg)
    C = jnp.where(neg, _scan2d(tneg, suffix=True), _scan2d(tpos) + total_neg)

    def pick(k):
        b_ord = jnp.sum((C < k).astype(jnp.int32))
        cb = jnp.max(jnp.where(C < k, C, 0))
        raw = jnp.where(b_ord < half, H1 - 1 - b_ord, b_ord - half)
        return raw, cb

    p_lo, cb_lo = pick(K_LO)
    p_hi, cb_hi = pick(K_HI)
    z = jnp.zeros((L,), jnp.int32)
    o_ref[...] = jnp.stack([
        jnp.full((L,), p_lo), jnp.full((L,), p_hi),
        jnp.full((L,), cb_lo), jnp.full((L,), cb_hi),
        z, z, z, z,
    ])


_glue1 = pl.pallas_call(
    _glue1_body, out_shape=jax.ShapeDtypeStruct((8, L), jnp.int32)
)


def _region_pick(cnt, is_neg, kp, n):
    C = jnp.where(is_neg, _scan2d(cnt, suffix=True), _scan2d(cnt))
    b_ord = jnp.sum((C < kp).astype(jnp.int32))
    cb = jnp.max(jnp.where(C < kp, C, 0))
    raw = jnp.where(is_neg, n - 1 - b_ord, b_ord)
    return raw, cb


def _refine_pick(h, pf, sign_shift, bits):
    n = 1 << bits
    R = n // 128
    t = jnp.sum(
        jnp.reshape(h[:, : 2 * n], (NW, 2, R, 128)), axis=0)
    r_lo = t[0]
    r_hi = t[1]

    pfx_lo = pf[0, 0]
    pfx_hi = pf[1, 0]
    cb1_lo = pf[2, 0]
    cb1_hi = pf[3, 0]
    r_hi = jnp.where(pfx_lo == pfx_hi, r_lo, r_hi)
    neg_lo = lax.shift_right_logical(pfx_lo, sign_shift) == 1
    neg_hi = lax.shift_right_logical(pfx_hi, sign_shift) == 1

    b_lo, cb2_lo = _region_pick(r_lo, neg_lo, K_LO - cb1_lo, n)
    b_hi, cb2_hi = _region_pick(r_hi, neg_hi, K_HI - cb1_hi, n)
    npfx_lo = lax.shift_left(pfx_lo, bits) | b_lo
    npfx_hi = lax.shift_left(pfx_hi, bits) | b_hi
    return npfx_lo, npfx_hi, cb1_lo + cb2_lo, cb1_hi + cb2_hi


def _glue2_body(h_ref, pf_ref, o_ref):
    npfx_lo, npfx_hi, ncb_lo, ncb_hi = _refine_pick(
        h_ref[...], pf_ref[...], sign_shift=B1 - 1, bits=B2)
    z = jnp.zeros((L,), jnp.int32)
    o_ref[...] = jnp.stack([
        jnp.full((L,), npfx_lo), jnp.full((L,), npfx_hi),
        jnp.full((L,), ncb_lo), jnp.full((L,), ncb_hi),
        z, z, z, z,
    ])


_glue2 = pl.pallas_call(
    _glue2_body, out_shape=jax.ShapeDtypeStruct((8, L), jnp.int32)
)


def _glue3_body(h_ref, pf_ref, w_ref, o_ref):
    key_lo, key_hi, _, _ = _refine_pick(
        h_ref[...], pf_ref[...], sign_shift=B1 + B2 - 1, bits=B3)

    lower_val = lax.bitcast_convert_type(key_lo, jnp.float32)
    upper_val = lax.bitcast_convert_type(key_hi, jnp.float32)

    w = w_ref[...]
    n = jnp.float32(w.size)
    w_abs_mean = jnp.sum(jnp.abs(w)) / n
    w_std = jnp.sqrt(jnp.sum(w * w) / n)
    w_clip = jnp.float32(-12.8) * w_abs_mean + jnp.float32(12.68) * w_std

    row = lax.broadcasted_iota(jnp.int32, (8, 128), 0)
    col = lax.broadcasted_iota(jnp.int32, (8, 128), 1)
    vals = jnp.where(
        col == 0, upper_val, jnp.where(col == 1, lower_val, w_clip)
    )
    o_ref[...] = jnp.where((row == 0) & (col < 3), vals, 0.0)


_glue3 = pl.pallas_call(
    _glue3_body, out_shape=jax.ShapeDtypeStruct((8, 128), jnp.float32)
)


def kernel(x, weight):
    _sc_pass1, _sc_pass2, _sc_pass3 = _build_sc_kernels()
    xf = jnp.reshape(x, (ROWS, COLS))
    h1 = _sc_pass1(xf)
    pf1 = _glue1(h1)
    h2 = _sc_pass2(xf, pf1)
    pf2 = _glue2(h2, pf1)
    h3 = _sc_pass3(xf, pf2)
    o = _glue3(h3, pf2, weight)
    return o[0, :3]
